# baseline (speedup 1.0000x reference)
import functools

import jax
import jax.numpy as jnp
from jax.experimental import pallas as pl
from jax.experimental.pallas import tpu as pltpu
from jax.experimental.pallas import tpu_sc as plsc

B = 16384
N_CAT = 26
VOCAB = 100000
EMB = 32
N_NUM = 13
CAT_H = 128
NUM_H = 128
HEAD_H = CAT_H + NUM_H
OUT = 64
EPS = 1e-5

N_TOTAL = B * N_CAT
SC_CORES = 2
SC_SUBCORES = 16
NW = SC_CORES * SC_SUBCORES
ROWS_W = B // NW
CAT_W = 896
N_FILL = (CAT_W - N_CAT * EMB) // EMB
BLK = 1024
NB = B // BLK


def _leaky(x):
    return jnp.where(x >= 0, x, 0.01 * x)


def _sc_gather(emb_tables, idx_T):
    mesh = plsc.VectorSubcoreMesh(core_axis_name="c", subcore_axis_name="s")

    @functools.partial(
        pl.kernel, mesh=mesh,
        out_type=jax.ShapeDtypeStruct((B, CAT_W), emb_tables.dtype),
        scratch_types=[
            pltpu.VMEM((ROWS_W,), jnp.int32),
            pltpu.VMEM((ROWS_W, EMB), jnp.float32),
            pltpu.SemaphoreType.DMA,
        ],
        compiler_params=pltpu.CompilerParams(use_tc_tiling_on_sc=False),
    )
    def kern(table_hbm, idx_hbm, out_hbm, idx_v, rows_v, sem):
        wid = jax.lax.axis_index("s") * SC_CORES + jax.lax.axis_index("c")
        b0 = wid * ROWS_W

        for ff in range(N_CAT + N_FILL):
            f = ff if ff < N_CAT else 0
            pltpu.sync_copy(idx_hbm.at[f, pl.ds(b0, ROWS_W)], idx_v)
            pltpu.async_copy(table_hbm.at[f].at[idx_v], rows_v, sem).wait()
            pltpu.sync_copy(rows_v,
                            out_hbm.at[pl.ds(b0, ROWS_W),
                                       pl.ds(ff * EMB, EMB)])

    return kern(emb_tables, idx_T)


def _num_tower_body(x_ref, w1_ref, b1_ref, w2_ref, b2_ref,
                    bn0g_ref, bn0b_ref, bn1g_ref, bn1b_ref,
                    bn2g_ref, bn2b_ref, o_ref):
    x = x_ref[...]
    mu = jnp.mean(x, axis=0, keepdims=True)
    var = jnp.mean((x - mu) ** 2, axis=0, keepdims=True)
    h = bn0g_ref[...] * (x - mu) / jnp.sqrt(var + EPS) + bn0b_ref[...]
    h = jnp.dot(h, w1_ref[...], preferred_element_type=jnp.float32) + b1_ref[...]
    mu = jnp.mean(h, axis=0, keepdims=True)
    var = jnp.mean((h - mu) ** 2, axis=0, keepdims=True)
    h = _leaky(bn1g_ref[...] * (h - mu) / jnp.sqrt(var + EPS) + bn1b_ref[...])
    h = jnp.dot(h, w2_ref[...], preferred_element_type=jnp.float32) + b2_ref[...]
    mu = jnp.mean(h, axis=0, keepdims=True)
    var = jnp.mean((h - mu) ** 2, axis=0, keepdims=True)
    o_ref[...] = _leaky(bn2g_ref[...] * (h - mu) / jnp.sqrt(var + EPS)
                        + bn2b_ref[...])


def _mid_body(g_ref, n_ref, wcat_ref, bcat_ref, lng_ref, lnb_ref,
              wh1_ref, bh1_ref, e1_ref, stats_ref):
    i = pl.program_id(0)
    c = jnp.dot(g_ref[...], wcat_ref[...],
                preferred_element_type=jnp.float32) + bcat_ref[...]
    mu = jnp.mean(c, axis=-1, keepdims=True)
    var = jnp.mean((c - mu) ** 2, axis=-1, keepdims=True)
    c = lng_ref[...] * (c - mu) / jnp.sqrt(var + EPS) + lnb_ref[...]
    e1 = (jnp.dot(n_ref[...], wh1_ref[0:NUM_H, :],
                  preferred_element_type=jnp.float32)
          + jnp.dot(c, wh1_ref[NUM_H:HEAD_H, :],
                    preferred_element_type=jnp.float32)
          + bh1_ref[...])
    e1_ref[...] = e1
    s = jnp.sum(e1, axis=0, keepdims=True)
    s2 = jnp.sum(e1 * e1, axis=0, keepdims=True)
    st = jnp.concatenate([s, s2], axis=0)

    @pl.when(i == 0)
    def _():
        stats_ref[...] = st

    @pl.when(i > 0)
    def _():
        stats_ref[...] += st


def _head_body(e1_ref, stats_ref, wh2_ref, bh2_ref, bnhg_ref, bnhb_ref, o_ref):
    st = stats_ref[...]
    mu = st[0:1, :] * (1.0 / B)
    var = st[1:2, :] * (1.0 / B) - mu * mu
    e = _leaky(bnhg_ref[...] * (e1_ref[...] - mu) / jnp.sqrt(var + EPS)
               + bnhb_ref[...])
    e = jnp.dot(e, wh2_ref[...], preferred_element_type=jnp.float32) + bh2_ref[...]
    o_ref[...] = e / jnp.sqrt(jnp.sum(e * e, axis=-1, keepdims=True))


def kernel(num_features, cat_features, emb_tables, W_cat, b_cat, ln_g, ln_b,
           bn0_g, bn0_b, W_n1, b_n1, bn1_g, bn1_b, W_n2, b_n2, bn2_g, bn2_b,
           W_h1, b_h1, bnh_g, bnh_b, W_h2, b_h2):
    f32 = jnp.float32
    r2 = lambda v: v.reshape(1, -1)

    idx_T = cat_features.astype(jnp.int32).T
    gathered = _sc_gather(emb_tables, idx_T)

    W_cat_pad = jnp.zeros((CAT_W, CAT_H), f32).at[0:N_CAT * EMB, :].set(W_cat)

    num_embs = pl.pallas_call(
        _num_tower_body,
        out_shape=jax.ShapeDtypeStruct((B, NUM_H), f32),
    )(num_features, W_n1, r2(b_n1), W_n2, r2(b_n2),
      r2(bn0_g), r2(bn0_b), r2(bn1_g), r2(bn1_b), r2(bn2_g), r2(bn2_b))

    row_blk = lambda i: (i, 0)
    whole = lambda i: (0, 0)
    e1, stats = pl.pallas_call(
        _mid_body,
        grid=(NB,),
        in_specs=[
            pl.BlockSpec((BLK, CAT_W), row_blk),
            pl.BlockSpec((BLK, NUM_H), row_blk),
            pl.BlockSpec((CAT_W, CAT_H), whole),
            pl.BlockSpec((1, CAT_H), whole),
            pl.BlockSpec((1, CAT_H), whole),
            pl.BlockSpec((1, CAT_H), whole),
            pl.BlockSpec((HEAD_H, HEAD_H), whole),
            pl.BlockSpec((1, HEAD_H), whole),
        ],
        out_specs=[
            pl.BlockSpec((BLK, HEAD_H), row_blk),
            pl.BlockSpec((2, HEAD_H), whole),
        ],
        out_shape=[
            jax.ShapeDtypeStruct((B, HEAD_H), f32),
            jax.ShapeDtypeStruct((2, HEAD_H), f32),
        ],
    )(gathered, num_embs, W_cat_pad, r2(b_cat), r2(ln_g), r2(ln_b),
      W_h1, r2(b_h1))

    out = pl.pallas_call(
        _head_body,
        grid=(NB,),
        in_specs=[
            pl.BlockSpec((BLK, HEAD_H), row_blk),
            pl.BlockSpec((2, HEAD_H), whole),
            pl.BlockSpec((HEAD_H, OUT), whole),
            pl.BlockSpec((1, OUT), whole),
            pl.BlockSpec((1, HEAD_H), whole),
            pl.BlockSpec((1, HEAD_H), whole),
        ],
        out_specs=pl.BlockSpec((BLK, OUT), row_blk),
        out_shape=jax.ShapeDtypeStruct((B, OUT), f32),
    )(e1, stats, W_h2, r2(b_h2), r2(bnh_g), r2(bnh_b))

    return out
